# baseline (speedup 1.0000x reference)
import functools

import jax
import jax.numpy as jnp
from jax import lax
from jax.experimental import pallas as pl
from jax.experimental.pallas import tpu as pltpu
from jax.experimental.pallas import tpu_sc as plsc


def kernel(x):
    B, C, H, W, Z = x.shape
    r = 2
    Ho, Wo, Zo = H // r, W // r, Z // r
    OC = C * r**3
    K = r**3
    NIN = 3
    NOUT = 2
    GRP = NIN * NOUT

    info = plsc.get_sparse_core_info()
    NC, NS, L = info.num_cores, info.num_subcores, info.num_lanes
    NW = NC * NS

    UNITS = B * C * Ho
    UPW = UNITS // NW
    VECS = (Wo * Zo) // L

    mesh = plsc.VectorSubcoreMesh(core_axis_name="c", subcore_axis_name="s")

    @functools.partial(
        pl.kernel,
        mesh=mesh,
        out_type=jax.ShapeDtypeStruct((B, OC, Ho, Wo, Zo), jnp.float32),
        scratch_types=(
            [pltpu.VMEM((NIN, r, W, Z), jnp.float32),
             pltpu.VMEM((NOUT, K, Wo, Zo), jnp.float32)]
            + [pltpu.SemaphoreType.DMA] * (NIN + NOUT)
        ),
        compiler_params=pltpu.CompilerParams(needs_layout_passes=False),
    )
    def body(x_hbm, y_hbm, in_ring, out_ring, *sems):
        in_sems = sems[:NIN]
        out_sems = sems[NIN:]
        wid = lax.axis_index("s") * NC + lax.axis_index("c")
        lane = lax.iota(jnp.int32, L)
        u0 = wid * UPW

        def unit_coords(t):
            u = u0 + t
            b = u // (C * Ho)
            rem = u % (C * Ho)
            c = rem // Ho
            ho = rem % Ho
            return b, c, ho

        def issue_in(t, j):
            b, c, ho = unit_coords(t)
            pltpu.make_async_copy(
                x_hbm.at[b, c, pl.ds(r * ho, r)],
                in_ring.at[j], in_sems[j]).start()

        def wait_in(j):
            pltpu.make_async_copy(
                x_hbm.at[0, 0, pl.ds(0, r)],
                in_ring.at[j], in_sems[j]).wait()

        def issue_out(t, j):
            b, c, ho = unit_coords(t)
            for k in range(K):
                pltpu.make_async_copy(
                    out_ring.at[j, k],
                    y_hbm.at[b, c * K + k, ho], out_sems[j]).start()

        def drain_out(j):
            for k in range(K):
                pltpu.make_async_copy(
                    out_ring.at[j, k],
                    y_hbm.at[0, k, 0], out_sems[j]).wait()

        def compute(ji, jo):
            in_b = in_ring.at[ji]

            def vec_body(v, carry):
                wo = v >> 1
                zo0 = (v & 1) * L
                idx_z0 = r * zo0 + lane * r
                for k in range(K):
                    hr, wr, zr = (k >> 2) & 1, (k >> 1) & 1, k & 1
                    vals = plsc.load_gather(
                        in_b,
                        [jnp.full((L,), hr, jnp.int32),
                         jnp.full((L,), r * wo + wr, jnp.int32),
                         idx_z0 + zr])
                    out_ring[jo, k, wo, pl.ds(zo0, L)] = vals
                return carry

            lax.fori_loop(0, VECS, vec_body, 0)

        for j in range(NIN):
            issue_in(j, j)

        def grp_body(p, carry):
            for j in range(GRP):
                t_traced = GRP * p + j
                ji, jo = j % NIN, j % NOUT
                wait_in(ji)

                @pl.when(t_traced >= NOUT)
                def _():
                    drain_out(jo)

                compute(ji, jo)
                issue_out(t_traced, jo)

                @pl.when(t_traced < UPW - NIN)
                def _():
                    issue_in(t_traced + NIN, ji)
            return carry

        NFULL = UPW // GRP
        lax.fori_loop(0, NFULL, grp_body, 0)
        for t in range(NFULL * GRP, UPW):
            j = t % GRP
            ji, jo = j % NIN, j % NOUT
            wait_in(ji)
            drain_out(jo)
            compute(ji, jo)
            issue_out(t, jo)
            if t + NIN < UPW:
                issue_in(t + NIN, ji)
        for t in range(UPW - NOUT, UPW):
            drain_out(t % NOUT)

    return body(x)


# speedup vs baseline: 1.0017x; 1.0017x over previous
import functools

import jax
import jax.numpy as jnp
from jax import lax
from jax.experimental import pallas as pl
from jax.experimental.pallas import tpu as pltpu
from jax.experimental.pallas import tpu_sc as plsc


def kernel(x):
    B, C, H, W, Z = x.shape
    r = 2
    Ho, Wo, Zo = H // r, W // r, Z // r
    OC = C * r**3
    K = r**3
    NIN = 3
    NOUT = 2
    GRP = NIN * NOUT

    info = plsc.get_sparse_core_info()
    NC, NS, L = info.num_cores, info.num_subcores, info.num_lanes
    NW = NC * NS

    UNITS = B * C * Ho
    UPW = UNITS // NW
    VECS = (Wo * Zo) // L

    mesh = plsc.VectorSubcoreMesh(core_axis_name="c", subcore_axis_name="s")

    @functools.partial(
        pl.kernel,
        mesh=mesh,
        out_type=jax.ShapeDtypeStruct((B, OC, Ho, Wo, Zo), jnp.float32),
        scratch_types=(
            [pltpu.VMEM((NIN, r, W, Z), jnp.float32),
             pltpu.VMEM((NOUT, K, Wo, Zo), jnp.float32)]
            + [pltpu.SemaphoreType.DMA] * (NIN + NOUT)
        ),
        compiler_params=pltpu.CompilerParams(needs_layout_passes=False),
    )
    def body(x_hbm, y_hbm, in_ring, out_ring, *sems):
        in_sems = sems[:NIN]
        out_sems = sems[NIN:]
        wid = lax.axis_index("s") * NC + lax.axis_index("c")
        lane = lax.iota(jnp.int32, L)
        u0 = wid * UPW

        def unit_coords(t):
            u = u0 + t
            b = u // (C * Ho)
            rem = u % (C * Ho)
            c = rem // Ho
            ho = rem % Ho
            return b, c, ho

        def issue_in(t, j):
            b, c, ho = unit_coords(t)
            pltpu.make_async_copy(
                x_hbm.at[b, c, pl.ds(r * ho, r)],
                in_ring.at[j], in_sems[j]).start()

        def wait_in(j):
            pltpu.make_async_copy(
                x_hbm.at[0, 0, pl.ds(0, r)],
                in_ring.at[j], in_sems[j]).wait()

        def issue_out(t, j):
            b, c, ho = unit_coords(t)
            pltpu.make_async_copy(
                out_ring.at[j],
                y_hbm.at[b, pl.ds(c * K, K), ho], out_sems[j]).start()

        def drain_out(j):
            pltpu.make_async_copy(
                out_ring.at[j],
                y_hbm.at[0, pl.ds(0, K), 0], out_sems[j]).wait()

        def compute(ji, jo):
            in_b = in_ring.at[ji]

            def vec_body(v, carry):
                wo = v >> 1
                zo0 = (v & 1) * L
                idx_z0 = r * zo0 + lane * r
                for k in range(K):
                    hr, wr, zr = (k >> 2) & 1, (k >> 1) & 1, k & 1
                    vals = plsc.load_gather(
                        in_b,
                        [jnp.full((L,), hr, jnp.int32),
                         jnp.full((L,), r * wo + wr, jnp.int32),
                         idx_z0 + zr])
                    out_ring[jo, k, wo, pl.ds(zo0, L)] = vals
                return carry

            lax.fori_loop(0, VECS, vec_body, 0)

        for j in range(NIN):
            issue_in(j, j)

        def grp_body(p, carry):
            for j in range(GRP):
                t_traced = GRP * p + j
                ji, jo = j % NIN, j % NOUT
                wait_in(ji)

                @pl.when(t_traced >= NOUT)
                def _():
                    drain_out(jo)

                compute(ji, jo)
                issue_out(t_traced, jo)

                @pl.when(t_traced < UPW - NIN)
                def _():
                    issue_in(t_traced + NIN, ji)
            return carry

        NFULL = UPW // GRP
        lax.fori_loop(0, NFULL, grp_body, 0)
        for t in range(NFULL * GRP, UPW):
            j = t % GRP
            ji, jo = j % NIN, j % NOUT
            wait_in(ji)
            drain_out(jo)
            compute(ji, jo)
            issue_out(t, jo)
            if t + NIN < UPW:
                issue_in(t + NIN, ji)
        for t in range(UPW - NOUT, UPW):
            drain_out(t % NOUT)

    return body(x)
